# baseline (speedup 1.0000x reference)
import functools

import jax
import jax.numpy as jnp
from jax import lax
from jax.experimental import pallas as pl
from jax.experimental.pallas import tpu as pltpu
from jax.experimental.pallas import tpu_sc as plsc

NUM_Q = 8
K = 8192
D = 256
T = 8192
TH = T // 2
BT_VQ = 512
NB_H = TH // BT_VQ
BT_MLP = 512

SC_CORES = 2
SC_SUBCORES = 16
NW = SC_CORES * SC_SUBCORES
GCH = TH // NW


def _enc_body(x_ref, w0_ref, b0_ref, w1_ref, b1_ref, z_ref):
    h = jnp.dot(x_ref[...], w0_ref[...], preferred_element_type=jnp.float32)
    h = jnp.maximum(h + b0_ref[...], 0.0)
    z_ref[...] = jnp.dot(h, w1_ref[...], preferred_element_type=jnp.float32) + b1_ref[...]


def _encoder(xf, w0, b0, w1, b1):
    nb = T // BT_MLP
    return pl.pallas_call(
        _enc_body,
        grid=(nb,),
        in_specs=[
            pl.BlockSpec((BT_MLP, 768), lambda i: (i, 0)),
            pl.BlockSpec((768, 512), lambda i: (0, 0)),
            pl.BlockSpec((1, 512), lambda i: (0, 0)),
            pl.BlockSpec((512, 256), lambda i: (0, 0)),
            pl.BlockSpec((1, 256), lambda i: (0, 0)),
        ],
        out_specs=pl.BlockSpec((BT_MLP, 256), lambda i: (i, 0)),
        out_shape=jax.ShapeDtypeStruct((T, 256), jnp.float32),
    )(xf, w0, b0, w1, b1)


def _make_vq_body(idx_off):
    def body(r_ref, cb_ref, cc_ref, idx_ref):
        r = r_ref[...]
        rc = lax.dot_general(r, cb_ref[0], (((1,), (1,)), ((), ())),
                             preferred_element_type=jnp.float32)
        rr = jnp.sum(r * r, axis=1, keepdims=True)
        d = (rr - 2.0 * rc) + cc_ref[0]
        idx_ref[0, 0, :] = jnp.argmin(d, axis=1).astype(jnp.int32) + idx_off
    return body


@functools.lru_cache(maxsize=None)
def _vq_argmin_call(layer, roff):
    return pl.pallas_call(
        _make_vq_body(layer * K),
        grid=(NB_H,),
        in_specs=[
            pl.BlockSpec((BT_VQ, D), lambda i: (i + roff, 0)),
            pl.BlockSpec((1, K, D), lambda i: (layer, 0, 0)),
            pl.BlockSpec((1, 1, K), lambda i: (layer, 0, 0)),
        ],
        out_specs=pl.BlockSpec((1, 1, BT_VQ), lambda i: (i, 0, 0)),
        out_shape=jax.ShapeDtypeStruct((NB_H, 1, BT_VQ), jnp.int32),
    )


def _sc_body(cb_hbm, idx_hbm, rp_hbm, out_hbm, idx_v, rows_v, rp_v, sem):
    wid = lax.axis_index("c") * SC_SUBCORES + lax.axis_index("s")
    pltpu.sync_copy(idx_hbm.at[pl.ds(wid, 1)], idx_v)
    gather = pltpu.async_copy(cb_hbm.at[idx_v.at[0]], rows_v, sem)
    pltpu.sync_copy(rp_hbm.at[pl.ds(wid * GCH, GCH)], rp_v)
    gather.wait()

    def row_fn(i, carry):
        for c in range(D // 16):
            sl = pl.ds(c * 16, 16)
            rp_v[i, sl] = rp_v[i, sl] - rows_v[i, sl]
        return carry

    lax.fori_loop(0, GCH, row_fn, 0)
    pltpu.sync_copy(rp_v, out_hbm.at[pl.ds(wid * GCH, GCH)])


@functools.lru_cache(maxsize=1)
def _sc_update_call():
    return functools.partial(
        pl.kernel,
        mesh=plsc.VectorSubcoreMesh(core_axis_name="c", subcore_axis_name="s",
                                    num_cores=SC_CORES),
        out_type=jax.ShapeDtypeStruct((TH, D), jnp.float32),
        scratch_types=[
            pltpu.VMEM((1, GCH), jnp.int32),
            pltpu.VMEM((GCH, D), jnp.float32),
            pltpu.VMEM((GCH, D), jnp.float32),
            pltpu.SemaphoreType.DMA,
        ],
    )(_sc_body)


def _sc_update(cb_flat, idx2, r_prev):
    return _sc_update_call()(cb_flat, idx2, r_prev)


def _dec_body(z_ref, ra_ref, rb_ref, w0_ref, b0_ref, w1_ref, b1_ref, out_ref):
    r = jnp.where(pl.program_id(0) < NB_H, ra_ref[...], rb_ref[...])
    q = z_ref[...] - r
    h = jnp.dot(q, w0_ref[...], preferred_element_type=jnp.float32)
    h = jnp.maximum(h + b0_ref[...], 0.0)
    out_ref[...] = jnp.dot(h, w1_ref[...], preferred_element_type=jnp.float32) + b1_ref[...]


def _decoder(z, ra, rb, w0, b0, w1, b1):
    nb = T // BT_MLP
    return pl.pallas_call(
        _dec_body,
        grid=(nb,),
        in_specs=[
            pl.BlockSpec((BT_MLP, 256), lambda i: (i, 0)),
            pl.BlockSpec((BT_MLP, 256), lambda i: (jnp.minimum(i, NB_H - 1), 0)),
            pl.BlockSpec((BT_MLP, 256), lambda i: (jnp.maximum(i - NB_H, 0), 0)),
            pl.BlockSpec((256, 512), lambda i: (0, 0)),
            pl.BlockSpec((1, 512), lambda i: (0, 0)),
            pl.BlockSpec((512, 768), lambda i: (0, 0)),
            pl.BlockSpec((1, 768), lambda i: (0, 0)),
        ],
        out_specs=pl.BlockSpec((BT_MLP, 768), lambda i: (i, 0)),
        out_shape=jax.ShapeDtypeStruct((T, 768), jnp.float32),
    )(z, ra, rb, w0, b0, w1, b1)


def kernel(x, enc_W0, enc_b0, enc_W1, enc_b1, dec_W0, dec_b0, dec_W1, dec_b1, codebooks):
    B, N, F = x.shape
    xf = x.reshape(T, F)
    z = _encoder(xf, enc_W0, enc_b0.reshape(1, -1), enc_W1, enc_b1.reshape(1, -1))

    cc_all = jnp.sum(codebooks ** 2, axis=-1).reshape(NUM_Q, 1, K)
    cb_flat = codebooks.reshape(NUM_Q * K, D)

    warm = _sc_update(cb_flat, jnp.zeros((NW, GCH), jnp.int32),
                      jnp.zeros((TH, D), jnp.float32))

    r = [z[:TH], z[TH:]]
    for i in range(NUM_Q):
        for h in range(2):
            if i == 0:
                idx = _vq_argmin_call(i, h * NB_H)(z, codebooks, cc_all)
            else:
                idx = _vq_argmin_call(i, 0)(r[h], codebooks, cc_all)
            r[h] = _sc_update(cb_flat, idx.reshape(NW, GCH), r[h])

    out = _decoder(z, r[0], r[1], dec_W0, dec_b0.reshape(1, -1),
                   dec_W1, dec_b1.reshape(1, -1))
    out = out * (warm[0, 0] * 0.0 + 1.0)
    return out.reshape(B, N, 768)


# speedup vs baseline: 1.1342x; 1.1342x over previous
import functools

import jax
import jax.numpy as jnp
from jax import lax
from jax.experimental import pallas as pl
from jax.experimental.pallas import tpu as pltpu
from jax.experimental.pallas import tpu_sc as plsc

NUM_Q = 8
K = 8192
D = 256
T = 8192
TH = T // 2
BT_VQ = 512
NB_H = TH // BT_VQ
BT_MLP = 512

SC_CORES = 2
SC_SUBCORES = 16
NW = SC_CORES * SC_SUBCORES
GCH = TH // NW


def _enc_body(x_ref, w0_ref, b0_ref, w1_ref, b1_ref, z_ref):
    h = jnp.dot(x_ref[...], w0_ref[...], preferred_element_type=jnp.float32)
    h = jnp.maximum(h + b0_ref[...], 0.0)
    z_ref[...] = jnp.dot(h, w1_ref[...], preferred_element_type=jnp.float32) + b1_ref[...]


def _encoder(xf, w0, b0, w1, b1):
    nb = T // BT_MLP
    return pl.pallas_call(
        _enc_body,
        grid=(nb,),
        in_specs=[
            pl.BlockSpec((BT_MLP, 768), lambda i: (i, 0)),
            pl.BlockSpec((768, 512), lambda i: (0, 0)),
            pl.BlockSpec((1, 512), lambda i: (0, 0)),
            pl.BlockSpec((512, 256), lambda i: (0, 0)),
            pl.BlockSpec((1, 256), lambda i: (0, 0)),
        ],
        out_specs=pl.BlockSpec((BT_MLP, 256), lambda i: (i, 0)),
        out_shape=jax.ShapeDtypeStruct((T, 256), jnp.float32),
    )(xf, w0, b0, w1, b1)


def _make_vq_body(idx_off):
    def body(r_ref, cb_ref, cc_ref, idx_ref):
        r = r_ref[...]
        rc = lax.dot_general(r, cb_ref[0], (((1,), (1,)), ((), ())),
                             preferred_element_type=jnp.float32)
        rr = jnp.sum(r * r, axis=1, keepdims=True)
        d = (rr - 2.0 * rc) + cc_ref[0]
        idx_ref[0, 0, :] = jnp.argmin(d, axis=1).astype(jnp.int32) + idx_off
    return body


@functools.lru_cache(maxsize=None)
def _vq_argmin_call(layer, roff):
    return pl.pallas_call(
        _make_vq_body(layer * K),
        grid=(NB_H,),
        in_specs=[
            pl.BlockSpec((BT_VQ, D), lambda i: (i + roff, 0)),
            pl.BlockSpec((1, K, D), lambda i: (layer, 0, 0)),
            pl.BlockSpec((1, 1, K), lambda i: (layer, 0, 0)),
        ],
        out_specs=pl.BlockSpec((1, 1, BT_VQ), lambda i: (i, 0, 0)),
        out_shape=jax.ShapeDtypeStruct((NB_H, 1, BT_VQ), jnp.int32),
    )


def _sc_body(cb_hbm, idx_hbm, rp_hbm, out_hbm, idx_v, rows_v, rp_v, sem):
    wid = lax.axis_index("c") * SC_SUBCORES + lax.axis_index("s")
    pltpu.sync_copy(idx_hbm.at[pl.ds(wid, 1)], idx_v)
    gather = pltpu.async_copy(cb_hbm.at[idx_v.at[0]], rows_v, sem)
    pltpu.sync_copy(rp_hbm.at[pl.ds(wid * GCH, GCH)], rp_v)
    gather.wait()

    def row_fn(i, carry):
        for c in range(D // 16):
            sl = pl.ds(c * 16, 16)
            rp_v[i, sl] = rp_v[i, sl] - rows_v[i, sl]
        return carry

    lax.fori_loop(0, GCH, row_fn, 0)
    pltpu.sync_copy(rp_v, out_hbm.at[pl.ds(wid * GCH, GCH)])


@functools.lru_cache(maxsize=1)
def _sc_update_call():
    return functools.partial(
        pl.kernel,
        mesh=plsc.VectorSubcoreMesh(core_axis_name="c", subcore_axis_name="s",
                                    num_cores=SC_CORES),
        out_type=jax.ShapeDtypeStruct((TH, D), jnp.float32),
        scratch_types=[
            pltpu.VMEM((1, GCH), jnp.int32),
            pltpu.VMEM((GCH, D), jnp.float32),
            pltpu.VMEM((GCH, D), jnp.float32),
            pltpu.SemaphoreType.DMA,
        ],
    )(_sc_body)


def _sc_update(cb_flat, idx2, r_prev):
    return _sc_update_call()(cb_flat, idx2, r_prev)


def _dec_body(z_ref, ra_ref, rb_ref, w0_ref, b0_ref, w1_ref, b1_ref, out_ref):
    r = jnp.where(pl.program_id(0) < NB_H, ra_ref[...], rb_ref[...])
    q = z_ref[...] - r
    h = jnp.dot(q, w0_ref[...], preferred_element_type=jnp.float32)
    h = jnp.maximum(h + b0_ref[...], 0.0)
    out_ref[...] = jnp.dot(h, w1_ref[...], preferred_element_type=jnp.float32) + b1_ref[...]


def _decoder(z, ra, rb, w0, b0, w1, b1):
    nb = T // BT_MLP
    return pl.pallas_call(
        _dec_body,
        grid=(nb,),
        in_specs=[
            pl.BlockSpec((BT_MLP, 256), lambda i: (i, 0)),
            pl.BlockSpec((BT_MLP, 256), lambda i: (jnp.minimum(i, NB_H - 1), 0)),
            pl.BlockSpec((BT_MLP, 256), lambda i: (jnp.maximum(i - NB_H, 0), 0)),
            pl.BlockSpec((256, 512), lambda i: (0, 0)),
            pl.BlockSpec((1, 512), lambda i: (0, 0)),
            pl.BlockSpec((512, 768), lambda i: (0, 0)),
            pl.BlockSpec((1, 768), lambda i: (0, 0)),
        ],
        out_specs=pl.BlockSpec((BT_MLP, 768), lambda i: (i, 0)),
        out_shape=jax.ShapeDtypeStruct((T, 768), jnp.float32),
    )(z, ra, rb, w0, b0, w1, b1)


def kernel(x, enc_W0, enc_b0, enc_W1, enc_b1, dec_W0, dec_b0, dec_W1, dec_b1, codebooks):
    B, N, F = x.shape
    xf = x.reshape(T, F)
    z = _encoder(xf, enc_W0, enc_b0.reshape(1, -1), enc_W1, enc_b1.reshape(1, -1))

    cc_all = jnp.sum(codebooks ** 2, axis=-1).reshape(NUM_Q, 1, K)
    cb_flat = codebooks.reshape(NUM_Q * K, D)

    r = [z[:TH], z[TH:]]
    for i in range(NUM_Q):
        for h in range(2):
            if i == 0:
                idx = _vq_argmin_call(i, h * NB_H)(z, codebooks, cc_all)
            else:
                idx = _vq_argmin_call(i, 0)(r[h], codebooks, cc_all)
            r[h] = _sc_update(cb_flat, idx.reshape(NW, GCH), r[h])

    out = _decoder(z, r[0], r[1], dec_W0, dec_b0.reshape(1, -1),
                   dec_W1, dec_b1.reshape(1, -1))
    return out.reshape(B, N, 768)


# speedup vs baseline: 1.2175x; 1.0734x over previous
import functools

import jax
import jax.numpy as jnp
from jax import lax
from jax.experimental import pallas as pl
from jax.experimental.pallas import tpu as pltpu
from jax.experimental.pallas import tpu_sc as plsc

NUM_Q = 8
K = 8192
D = 256
T = 8192
TH = T // 2
BT_VQ = 512
NB_H = TH // BT_VQ
BT_MLP = 512

SC_CORES = 2
SC_SUBCORES = 16
NW = SC_CORES * SC_SUBCORES
GCH = TH // NW


def _enc_body(x_ref, w0_ref, b0_ref, w1_ref, b1_ref, z_ref):
    h = jnp.dot(x_ref[...], w0_ref[...], preferred_element_type=jnp.float32)
    h = jnp.maximum(h + b0_ref[...], 0.0)
    z_ref[...] = jnp.dot(h, w1_ref[...], preferred_element_type=jnp.float32) + b1_ref[...]


def _encoder(xf, w0, b0, w1, b1):
    nb = T // BT_MLP
    return pl.pallas_call(
        _enc_body,
        grid=(nb,),
        in_specs=[
            pl.BlockSpec((BT_MLP, 768), lambda i: (i, 0)),
            pl.BlockSpec((768, 512), lambda i: (0, 0)),
            pl.BlockSpec((1, 512), lambda i: (0, 0)),
            pl.BlockSpec((512, 256), lambda i: (0, 0)),
            pl.BlockSpec((1, 256), lambda i: (0, 0)),
        ],
        out_specs=pl.BlockSpec((BT_MLP, 256), lambda i: (i, 0)),
        out_shape=jax.ShapeDtypeStruct((T, 256), jnp.float32),
    )(xf, w0, b0, w1, b1)


def _make_vq_body(idx_off):
    def body(r_ref, cb_ref, cc_ref, idx_ref):
        r = r_ref[...]
        rc2 = lax.dot_general(r * -2.0, cb_ref[0], (((1,), (1,)), ((), ())),
                              preferred_element_type=jnp.float32)
        rr = jnp.sum(r * r, axis=1, keepdims=True)
        d = (rr + rc2) + cc_ref[0]
        idx_ref[0, 0, :] = jnp.argmin(d, axis=1).astype(jnp.int32) + idx_off
    return body


@functools.lru_cache(maxsize=None)
def _vq_argmin_call(layer, roff):
    return pl.pallas_call(
        _make_vq_body(layer * K),
        grid=(NB_H,),
        in_specs=[
            pl.BlockSpec((BT_VQ, D), lambda i: (i + roff, 0)),
            pl.BlockSpec((1, K, D), lambda i: (layer, 0, 0)),
            pl.BlockSpec((1, 1, K), lambda i: (layer, 0, 0)),
        ],
        out_specs=pl.BlockSpec((1, 1, BT_VQ), lambda i: (i, 0, 0)),
        out_shape=jax.ShapeDtypeStruct((NB_H, 1, BT_VQ), jnp.int32),
    )


def _sc_body(cb_hbm, idx_hbm, rp_hbm, out_hbm, idx_v, rows_v, rp_v, sem):
    wid = lax.axis_index("c") * SC_SUBCORES + lax.axis_index("s")
    pltpu.sync_copy(idx_hbm.at[pl.ds(wid, 1)], idx_v)
    gather = pltpu.async_copy(cb_hbm.at[idx_v.at[0]], rows_v, sem)
    pltpu.sync_copy(rp_hbm.at[pl.ds(wid * GCH, GCH)], rp_v)
    gather.wait()

    def row_fn(i, carry):
        for c in range(D // 16):
            sl = pl.ds(c * 16, 16)
            rp_v[i, sl] = rp_v[i, sl] - rows_v[i, sl]
        return carry

    lax.fori_loop(0, GCH, row_fn, 0)
    pltpu.sync_copy(rp_v, out_hbm.at[pl.ds(wid * GCH, GCH)])


@functools.lru_cache(maxsize=1)
def _sc_update_call():
    return functools.partial(
        pl.kernel,
        mesh=plsc.VectorSubcoreMesh(core_axis_name="c", subcore_axis_name="s",
                                    num_cores=SC_CORES),
        out_type=jax.ShapeDtypeStruct((TH, D), jnp.float32),
        scratch_types=[
            pltpu.VMEM((1, GCH), jnp.int32),
            pltpu.VMEM((GCH, D), jnp.float32),
            pltpu.VMEM((GCH, D), jnp.float32),
            pltpu.SemaphoreType.DMA,
        ],
    )(_sc_body)


def _sc_update(cb_flat, idx2, r_prev):
    return _sc_update_call()(cb_flat, idx2, r_prev)


def _dec_body(z_ref, ra_ref, rb_ref, w0_ref, b0_ref, w1_ref, b1_ref, out_ref):
    r = jnp.where(pl.program_id(0) < NB_H, ra_ref[...], rb_ref[...])
    q = z_ref[...] - r
    h = jnp.dot(q, w0_ref[...], preferred_element_type=jnp.float32)
    h = jnp.maximum(h + b0_ref[...], 0.0)
    out_ref[...] = jnp.dot(h, w1_ref[...], preferred_element_type=jnp.float32) + b1_ref[...]


def _decoder(z, ra, rb, w0, b0, w1, b1):
    nb = T // BT_MLP
    return pl.pallas_call(
        _dec_body,
        grid=(nb,),
        in_specs=[
            pl.BlockSpec((BT_MLP, 256), lambda i: (i, 0)),
            pl.BlockSpec((BT_MLP, 256), lambda i: (jnp.minimum(i, NB_H - 1), 0)),
            pl.BlockSpec((BT_MLP, 256), lambda i: (jnp.maximum(i - NB_H, 0), 0)),
            pl.BlockSpec((256, 512), lambda i: (0, 0)),
            pl.BlockSpec((1, 512), lambda i: (0, 0)),
            pl.BlockSpec((512, 768), lambda i: (0, 0)),
            pl.BlockSpec((1, 768), lambda i: (0, 0)),
        ],
        out_specs=pl.BlockSpec((BT_MLP, 768), lambda i: (i, 0)),
        out_shape=jax.ShapeDtypeStruct((T, 768), jnp.float32),
    )(z, ra, rb, w0, b0, w1, b1)


def kernel(x, enc_W0, enc_b0, enc_W1, enc_b1, dec_W0, dec_b0, dec_W1, dec_b1, codebooks):
    B, N, F = x.shape
    xf = x.reshape(T, F)
    z = _encoder(xf, enc_W0, enc_b0.reshape(1, -1), enc_W1, enc_b1.reshape(1, -1))

    cc_all = jnp.sum(codebooks ** 2, axis=-1).reshape(NUM_Q, 1, K)
    cb_flat = codebooks.reshape(NUM_Q * K, D)

    warm_idx = lax.broadcasted_iota(jnp.int32, (NW, GCH), 1) * NW \
        + lax.broadcasted_iota(jnp.int32, (NW, GCH), 0)
    warm = _sc_update(cb_flat, warm_idx, jnp.zeros((TH, D), jnp.float32))

    r = [z[:TH], z[TH:]]
    for i in range(NUM_Q):
        for h in range(2):
            if i == 0:
                idx = _vq_argmin_call(i, h * NB_H)(z, codebooks, cc_all)
            else:
                idx = _vq_argmin_call(i, 0)(r[h], codebooks, cc_all)
            r[h] = _sc_update(cb_flat, idx.reshape(NW, GCH), r[h])

    out = _decoder(z, r[0], r[1], dec_W0, dec_b0.reshape(1, -1),
                   dec_W1, dec_b1.reshape(1, -1))
    out = out * (warm[0, 0] * 0.0 + 1.0)
    return out.reshape(B, N, 768)


# speedup vs baseline: 1.2471x; 1.0243x over previous
import functools

import jax
import jax.numpy as jnp
from jax import lax
from jax.experimental import pallas as pl
from jax.experimental.pallas import tpu as pltpu
from jax.experimental.pallas import tpu_sc as plsc

NUM_Q = 8
K = 8192
D = 256
T = 8192
TH = T // 2
BT_VQ = 512
NB_H = TH // BT_VQ
BT_MLP = 512

SC_CORES = 2
SC_SUBCORES = 16
NW = SC_CORES * SC_SUBCORES
GCH = TH // NW


def _enc_body(x_ref, w0_ref, b0_ref, w1_ref, b1_ref, z_ref):
    h = jnp.dot(x_ref[...], w0_ref[...], preferred_element_type=jnp.float32)
    h = jnp.maximum(h + b0_ref[...], 0.0)
    z_ref[...] = jnp.dot(h, w1_ref[...], preferred_element_type=jnp.float32) + b1_ref[...]


def _encoder(xf, w0, b0, w1, b1):
    nb = T // BT_MLP
    return pl.pallas_call(
        _enc_body,
        grid=(nb,),
        in_specs=[
            pl.BlockSpec((BT_MLP, 768), lambda i: (i, 0)),
            pl.BlockSpec((768, 512), lambda i: (0, 0)),
            pl.BlockSpec((1, 512), lambda i: (0, 0)),
            pl.BlockSpec((512, 256), lambda i: (0, 0)),
            pl.BlockSpec((1, 256), lambda i: (0, 0)),
        ],
        out_specs=pl.BlockSpec((BT_MLP, 256), lambda i: (i, 0)),
        out_shape=jax.ShapeDtypeStruct((T, 256), jnp.float32),
    )(xf, w0, b0, w1, b1)


def _make_vq_body(idx_off):
    def body(r_ref, cb_ref, cc_ref, idx_ref):
        r = r_ref[...]
        rc2 = lax.dot_general(r * -2.0, cb_ref[0], (((1,), (1,)), ((), ())),
                              preferred_element_type=jnp.float32)
        rr = jnp.sum(r * r, axis=1, keepdims=True)
        d = (rr + rc2) + cc_ref[0]
        idx_ref[0, 0, :] = jnp.argmin(d, axis=1).astype(jnp.int32) + idx_off
    return body


@functools.lru_cache(maxsize=None)
def _vq_argmin_call(layer, roff):
    return pl.pallas_call(
        _make_vq_body(layer * K),
        grid=(NB_H,),
        in_specs=[
            pl.BlockSpec((BT_VQ, D), lambda i: (i + roff, 0)),
            pl.BlockSpec((1, K, D), lambda i: (layer, 0, 0)),
            pl.BlockSpec((1, 1, K), lambda i: (layer, 0, 0)),
        ],
        out_specs=pl.BlockSpec((1, 1, BT_VQ), lambda i: (i, 0, 0)),
        out_shape=jax.ShapeDtypeStruct((NB_H, 1, BT_VQ), jnp.int32),
    )


def _sc_body(cb_hbm, idx_hbm, rp_hbm, out_hbm, idx_v, rows_v, rp_v, sem):
    wid = lax.axis_index("c") * SC_SUBCORES + lax.axis_index("s")
    pltpu.sync_copy(idx_hbm.at[pl.ds(wid, 1)], idx_v)
    gather = pltpu.async_copy(cb_hbm.at[idx_v.at[0]], rows_v, sem)
    pltpu.sync_copy(rp_hbm.at[pl.ds(wid * GCH, GCH)], rp_v)
    gather.wait()

    def row_fn(i, carry):
        for c in range(D // 16):
            sl = pl.ds(c * 16, 16)
            rp_v[i, sl] = rp_v[i, sl] - rows_v[i, sl]
        return carry

    lax.fori_loop(0, GCH, row_fn, 0)
    pltpu.sync_copy(rp_v, out_hbm.at[pl.ds(wid * GCH, GCH)])


@functools.lru_cache(maxsize=1)
def _sc_update_call():
    return functools.partial(
        pl.kernel,
        mesh=plsc.VectorSubcoreMesh(core_axis_name="c", subcore_axis_name="s",
                                    num_cores=SC_CORES),
        out_type=jax.ShapeDtypeStruct((TH, D), jnp.float32),
        scratch_types=[
            pltpu.VMEM((1, GCH), jnp.int32),
            pltpu.VMEM((GCH, D), jnp.float32),
            pltpu.VMEM((GCH, D), jnp.float32),
            pltpu.SemaphoreType.DMA,
        ],
    )(_sc_body)


def _sc_update(cb_flat, idx2, r_prev):
    return _sc_update_call()(cb_flat, idx2, r_prev)


def _dec_body(z_ref, ra_ref, rb_ref, w0_ref, b0_ref, w1_ref, b1_ref, out_ref):
    r = jnp.where(pl.program_id(0) < NB_H, ra_ref[...], rb_ref[...])
    q = z_ref[...] - r
    h = jnp.dot(q, w0_ref[...], preferred_element_type=jnp.float32)
    h = jnp.maximum(h + b0_ref[...], 0.0)
    out_ref[...] = jnp.dot(h, w1_ref[...], preferred_element_type=jnp.float32) + b1_ref[...]


def _decoder(z, ra, rb, w0, b0, w1, b1):
    nb = T // BT_MLP
    return pl.pallas_call(
        _dec_body,
        grid=(nb,),
        in_specs=[
            pl.BlockSpec((BT_MLP, 256), lambda i: (i, 0)),
            pl.BlockSpec((BT_MLP, 256), lambda i: (jnp.minimum(i, NB_H - 1), 0)),
            pl.BlockSpec((BT_MLP, 256), lambda i: (jnp.maximum(i - NB_H, 0), 0)),
            pl.BlockSpec((256, 512), lambda i: (0, 0)),
            pl.BlockSpec((1, 512), lambda i: (0, 0)),
            pl.BlockSpec((512, 768), lambda i: (0, 0)),
            pl.BlockSpec((1, 768), lambda i: (0, 0)),
        ],
        out_specs=pl.BlockSpec((BT_MLP, 768), lambda i: (i, 0)),
        out_shape=jax.ShapeDtypeStruct((T, 768), jnp.float32),
    )(z, ra, rb, w0, b0, w1, b1)


def kernel(x, enc_W0, enc_b0, enc_W1, enc_b1, dec_W0, dec_b0, dec_W1, dec_b1, codebooks):
    B, N, F = x.shape
    xf = x.reshape(T, F)
    z = _encoder(xf, enc_W0, enc_b0.reshape(1, -1), enc_W1, enc_b1.reshape(1, -1))

    cc_all = jnp.sum(codebooks ** 2, axis=-1).reshape(NUM_Q, 1, K)
    cb_flat = codebooks.reshape(NUM_Q * K, D)

    r = [z[:TH], z[TH:]]
    for i in range(NUM_Q):
        for h in range(2):
            if i == 0:
                idx = _vq_argmin_call(i, h * NB_H)(z, codebooks, cc_all)
            else:
                idx = _vq_argmin_call(i, 0)(r[h], codebooks, cc_all)
            r[h] = _sc_update(cb_flat, idx.reshape(NW, GCH), r[h])

    out = _decoder(z, r[0], r[1], dec_W0, dec_b0.reshape(1, -1),
                   dec_W1, dec_b1.reshape(1, -1))
    return out.reshape(B, N, 768)


# speedup vs baseline: 1.2522x; 1.0041x over previous
import functools

import jax
import jax.numpy as jnp
from jax import lax
from jax.experimental import pallas as pl
from jax.experimental.pallas import tpu as pltpu
from jax.experimental.pallas import tpu_sc as plsc

NUM_Q = 8
K = 8192
D = 256
T = 8192
TH = T // 2
TQ = T // 4
BT_VQ = 512
NB_H = TH // BT_VQ
NB_Q = TQ // BT_VQ
BT_MLP = 512

SC_CORES = 2
SC_SUBCORES = 16
NW = SC_CORES * SC_SUBCORES
GCH = TH // NW
GCQ = TQ // NW


def _enc_body(x_ref, w0_ref, b0_ref, w1_ref, b1_ref, z_ref):
    h = jnp.dot(x_ref[...], w0_ref[...], preferred_element_type=jnp.float32)
    h = jnp.maximum(h + b0_ref[...], 0.0)
    z_ref[...] = jnp.dot(h, w1_ref[...], preferred_element_type=jnp.float32) + b1_ref[...]


def _encoder(xf, w0, b0, w1, b1):
    nb = T // BT_MLP
    return pl.pallas_call(
        _enc_body,
        grid=(nb,),
        in_specs=[
            pl.BlockSpec((BT_MLP, 768), lambda i: (i, 0)),
            pl.BlockSpec((768, 512), lambda i: (0, 0)),
            pl.BlockSpec((1, 512), lambda i: (0, 0)),
            pl.BlockSpec((512, 256), lambda i: (0, 0)),
            pl.BlockSpec((1, 256), lambda i: (0, 0)),
        ],
        out_specs=pl.BlockSpec((BT_MLP, 256), lambda i: (i, 0)),
        out_shape=jax.ShapeDtypeStruct((T, 256), jnp.float32),
    )(xf, w0, b0, w1, b1)


def _argmin_of(r, cb, cc):
    rc2 = lax.dot_general(r * -2.0, cb, (((1,), (1,)), ((), ())),
                          preferred_element_type=jnp.float32)
    rr = jnp.sum(r * r, axis=1, keepdims=True)
    d = (rr + rc2) + cc
    return jnp.argmin(d, axis=1).astype(jnp.int32)


def _make_vq_body(idx_off):
    def body(r_ref, cb_ref, cc_ref, idx_ref):
        idx_ref[0, 0, :] = _argmin_of(r_ref[...], cb_ref[0], cc_ref[0]) + idx_off
    return body


@functools.lru_cache(maxsize=None)
def _vq_argmin_call(layer, cc_idx, roff, nb):
    return pl.pallas_call(
        _make_vq_body(layer * K),
        grid=(nb,),
        in_specs=[
            pl.BlockSpec((BT_VQ, D), lambda i: (i + roff, 0)),
            pl.BlockSpec((1, K, D), lambda i: (layer, 0, 0)),
            pl.BlockSpec((1, 1, K), lambda i: (cc_idx, 0, 0)),
        ],
        out_specs=pl.BlockSpec((1, 1, BT_VQ), lambda i: (i, 0, 0)),
        out_shape=jax.ShapeDtypeStruct((nb, 1, BT_VQ), jnp.int32),
    )


def _vq_trans_body(rqa_ref, rqb_ref, cb_ref, cc_ref, idx_ref, r_ref):
    r = jnp.where(pl.program_id(0) < NB_Q, rqa_ref[...], rqb_ref[...])
    r_ref[...] = r
    idx_ref[0, 0, :] = _argmin_of(r, cb_ref[0], cc_ref[0]) + K


@functools.lru_cache(maxsize=1)
def _vq_trans_call():
    return pl.pallas_call(
        _vq_trans_body,
        grid=(NB_H,),
        in_specs=[
            pl.BlockSpec((BT_VQ, D), lambda i: (jnp.minimum(i, NB_Q - 1), 0)),
            pl.BlockSpec((BT_VQ, D), lambda i: (jnp.maximum(i - NB_Q, 0), 0)),
            pl.BlockSpec((1, K, D), lambda i: (1, 0, 0)),
            pl.BlockSpec((1, 1, K), lambda i: (0, 0, 0)),
        ],
        out_specs=[
            pl.BlockSpec((1, 1, BT_VQ), lambda i: (i, 0, 0)),
            pl.BlockSpec((BT_VQ, D), lambda i: (i, 0)),
        ],
        out_shape=[
            jax.ShapeDtypeStruct((NB_H, 1, BT_VQ), jnp.int32),
            jax.ShapeDtypeStruct((TH, D), jnp.float32),
        ],
    )


def _make_sc_body(gch, roff):
    def body(cb_hbm, idx_hbm, rp_hbm, out_hbm, idx_v, rows_v, rp_v, sem):
        wid = lax.axis_index("c") * SC_SUBCORES + lax.axis_index("s")
        pltpu.sync_copy(idx_hbm.at[pl.ds(wid, 1)], idx_v)
        gather = pltpu.async_copy(cb_hbm.at[idx_v.at[0]], rows_v, sem)
        pltpu.sync_copy(rp_hbm.at[pl.ds(roff + wid * gch, gch)], rp_v)
        gather.wait()

        def row_fn(i, carry):
            for c in range(D // 16):
                sl = pl.ds(c * 16, 16)
                rp_v[i, sl] = rp_v[i, sl] - rows_v[i, sl]
            return carry

        lax.fori_loop(0, gch, row_fn, 0)
        pltpu.sync_copy(rp_v, out_hbm.at[pl.ds(wid * gch, gch)])
    return body


@functools.lru_cache(maxsize=None)
def _sc_update_call(nt, gch, roff):
    return functools.partial(
        pl.kernel,
        mesh=plsc.VectorSubcoreMesh(core_axis_name="c", subcore_axis_name="s",
                                    num_cores=SC_CORES),
        out_type=jax.ShapeDtypeStruct((nt, D), jnp.float32),
        scratch_types=[
            pltpu.VMEM((1, gch), jnp.int32),
            pltpu.VMEM((gch, D), jnp.float32),
            pltpu.VMEM((gch, D), jnp.float32),
            pltpu.SemaphoreType.DMA,
        ],
    )(_make_sc_body(gch, roff))


def _sc_update(cb_flat, idx2, r_prev, nt=TH, roff=0):
    return _sc_update_call(nt, idx2.shape[1], roff)(cb_flat, idx2, r_prev)


def _dec_body(z_ref, ra_ref, rb_ref, w0_ref, b0_ref, w1_ref, b1_ref, out_ref):
    r = jnp.where(pl.program_id(0) < NB_H, ra_ref[...], rb_ref[...])
    q = z_ref[...] - r
    h = jnp.dot(q, w0_ref[...], preferred_element_type=jnp.float32)
    h = jnp.maximum(h + b0_ref[...], 0.0)
    out_ref[...] = jnp.dot(h, w1_ref[...], preferred_element_type=jnp.float32) + b1_ref[...]


def _decoder(z, ra, rb, w0, b0, w1, b1):
    nb = T // BT_MLP
    return pl.pallas_call(
        _dec_body,
        grid=(nb,),
        in_specs=[
            pl.BlockSpec((BT_MLP, 256), lambda i: (i, 0)),
            pl.BlockSpec((BT_MLP, 256), lambda i: (jnp.minimum(i, NB_H - 1), 0)),
            pl.BlockSpec((BT_MLP, 256), lambda i: (jnp.maximum(i - NB_H, 0), 0)),
            pl.BlockSpec((256, 512), lambda i: (0, 0)),
            pl.BlockSpec((1, 512), lambda i: (0, 0)),
            pl.BlockSpec((512, 768), lambda i: (0, 0)),
            pl.BlockSpec((1, 768), lambda i: (0, 0)),
        ],
        out_specs=pl.BlockSpec((BT_MLP, 768), lambda i: (i, 0)),
        out_shape=jax.ShapeDtypeStruct((T, 768), jnp.float32),
    )(z, ra, rb, w0, b0, w1, b1)


def kernel(x, enc_W0, enc_b0, enc_W1, enc_b1, dec_W0, dec_b0, dec_W1, dec_b1, codebooks):
    B, N, F = x.shape
    xf = x.reshape(T, F)
    z = _encoder(xf, enc_W0, enc_b0.reshape(1, -1), enc_W1, enc_b1.reshape(1, -1))

    cc0 = jnp.sum(codebooks[0] ** 2, axis=-1).reshape(1, 1, K)
    ccR = jnp.sum(codebooks[1:] ** 2, axis=-1).reshape(NUM_Q - 1, 1, K)
    cb_flat = codebooks.reshape(NUM_Q * K, D)

    rq = []
    for qt in range(4):
        idx = _vq_argmin_call(0, 0, qt * NB_Q, NB_Q)(z, codebooks, cc0)
        rq.append(_sc_update(cb_flat, idx.reshape(NW, GCQ), z,
                             nt=TQ, roff=qt * TQ))

    r = [None, None]
    for h in range(2):
        idx, rh = _vq_trans_call()(rq[2 * h], rq[2 * h + 1], codebooks, ccR)
        r[h] = _sc_update(cb_flat, idx.reshape(NW, GCH), rh)

    for i in range(2, NUM_Q):
        for h in range(2):
            idx = _vq_argmin_call(i, i - 1, 0, NB_H)(r[h], codebooks, ccR)
            r[h] = _sc_update(cb_flat, idx.reshape(NW, GCH), r[h])

    out = _decoder(z, r[0], r[1], dec_W0, dec_b0.reshape(1, -1),
                   dec_W1, dec_b1.reshape(1, -1))
    return out.reshape(B, N, 768)


# speedup vs baseline: 1.2549x; 1.0022x over previous
import functools

import jax
import jax.numpy as jnp
from jax import lax
from jax.experimental import pallas as pl
from jax.experimental.pallas import tpu as pltpu
from jax.experimental.pallas import tpu_sc as plsc

NUM_Q = 8
K = 8192
D = 256
T = 8192
TH = T // 2
TQ = T // 4
BT_VQ = 512
NB_H = TH // BT_VQ
NB_Q = TQ // BT_VQ
BT_MLP = 512

SC_CORES = 2
SC_SUBCORES = 16
NW = SC_CORES * SC_SUBCORES
GCH = TH // NW
GCQ = TQ // NW


def _enc_body(x_ref, w0_ref, b0_ref, w1_ref, b1_ref, z_ref):
    h = jnp.dot(x_ref[...], w0_ref[...], preferred_element_type=jnp.float32)
    h = jnp.maximum(h + b0_ref[...], 0.0)
    z_ref[...] = jnp.dot(h, w1_ref[...], preferred_element_type=jnp.float32) + b1_ref[...]


def _encoder(xf, w0, b0, w1, b1):
    nb = T // BT_MLP
    return pl.pallas_call(
        _enc_body,
        grid=(nb,),
        in_specs=[
            pl.BlockSpec((BT_MLP, 768), lambda i: (i, 0)),
            pl.BlockSpec((768, 512), lambda i: (0, 0)),
            pl.BlockSpec((1, 512), lambda i: (0, 0)),
            pl.BlockSpec((512, 256), lambda i: (0, 0)),
            pl.BlockSpec((1, 256), lambda i: (0, 0)),
        ],
        out_specs=pl.BlockSpec((BT_MLP, 256), lambda i: (i, 0)),
        out_shape=jax.ShapeDtypeStruct((T, 256), jnp.float32),
    )(xf, w0, b0, w1, b1)


def _argmin_of(r, cb, cc):
    rc2 = lax.dot_general(r * -2.0, cb, (((1,), (1,)), ((), ())),
                          preferred_element_type=jnp.float32)
    rr = jnp.sum(r * r, axis=1, keepdims=True)
    d = (rr + rc2) + cc
    return jnp.argmin(d, axis=1).astype(jnp.int32)


def _make_vq_body(idx_off):
    def body(r_ref, cb_ref, cc_ref, idx_ref):
        idx_ref[0, 0, :] = _argmin_of(r_ref[...], cb_ref[0], cc_ref[0]) + idx_off
    return body


@functools.lru_cache(maxsize=None)
def _vq_argmin_call(layer, cc_idx, roff, nb):
    return pl.pallas_call(
        _make_vq_body(layer * K),
        grid=(nb,),
        in_specs=[
            pl.BlockSpec((BT_VQ, D), lambda i: (i + roff, 0)),
            pl.BlockSpec((1, K, D), lambda i: (layer, 0, 0)),
            pl.BlockSpec((1, 1, K), lambda i: (cc_idx, 0, 0)),
        ],
        out_specs=pl.BlockSpec((1, 1, BT_VQ), lambda i: (i, 0, 0)),
        out_shape=jax.ShapeDtypeStruct((nb, 1, BT_VQ), jnp.int32),
    )


def _vq_trans_body(rqa_ref, rqb_ref, cb_ref, cc_ref, idx_ref, r_ref):
    r = jnp.where(pl.program_id(0) < NB_Q, rqa_ref[...], rqb_ref[...])
    r_ref[...] = r
    idx_ref[0, 0, :] = _argmin_of(r, cb_ref[0], cc_ref[0]) + K


@functools.lru_cache(maxsize=1)
def _vq_trans_call():
    return pl.pallas_call(
        _vq_trans_body,
        grid=(NB_H,),
        in_specs=[
            pl.BlockSpec((BT_VQ, D), lambda i: (jnp.minimum(i, NB_Q - 1), 0)),
            pl.BlockSpec((BT_VQ, D), lambda i: (jnp.maximum(i - NB_Q, 0), 0)),
            pl.BlockSpec((1, K, D), lambda i: (1, 0, 0)),
            pl.BlockSpec((1, 1, K), lambda i: (1, 0, 0)),
        ],
        out_specs=[
            pl.BlockSpec((1, 1, BT_VQ), lambda i: (i, 0, 0)),
            pl.BlockSpec((BT_VQ, D), lambda i: (i, 0)),
        ],
        out_shape=[
            jax.ShapeDtypeStruct((NB_H, 1, BT_VQ), jnp.int32),
            jax.ShapeDtypeStruct((TH, D), jnp.float32),
        ],
    )


def _make_sc_body(gch, roff):
    def body(cb_hbm, idx_hbm, rp_hbm, out_hbm, idx_v, rows_v, rp_v, sem):
        wid = lax.axis_index("c") * SC_SUBCORES + lax.axis_index("s")
        pltpu.sync_copy(idx_hbm.at[pl.ds(wid, 1)], idx_v)
        gather = pltpu.async_copy(cb_hbm.at[idx_v.at[0]], rows_v, sem)
        pltpu.sync_copy(rp_hbm.at[pl.ds(roff + wid * gch, gch)], rp_v)
        gather.wait()

        def row_fn(i, carry):
            for c in range(D // 16):
                sl = pl.ds(c * 16, 16)
                rp_v[i, sl] = rp_v[i, sl] - rows_v[i, sl]
            return carry

        lax.fori_loop(0, gch, row_fn, 0)
        pltpu.sync_copy(rp_v, out_hbm.at[pl.ds(wid * gch, gch)])
    return body


@functools.lru_cache(maxsize=None)
def _sc_update_call(nt, gch, roff):
    return functools.partial(
        pl.kernel,
        mesh=plsc.VectorSubcoreMesh(core_axis_name="c", subcore_axis_name="s",
                                    num_cores=SC_CORES),
        out_type=jax.ShapeDtypeStruct((nt, D), jnp.float32),
        scratch_types=[
            pltpu.VMEM((1, gch), jnp.int32),
            pltpu.VMEM((gch, D), jnp.float32),
            pltpu.VMEM((gch, D), jnp.float32),
            pltpu.SemaphoreType.DMA,
        ],
    )(_make_sc_body(gch, roff))


def _sc_update(cb_flat, idx2, r_prev, nt=TH, roff=0):
    return _sc_update_call(nt, idx2.shape[1], roff)(cb_flat, idx2, r_prev)


def _dec_body(z_ref, ra_ref, rb_ref, w0_ref, b0_ref, w1_ref, b1_ref, out_ref):
    r = jnp.where(pl.program_id(0) < NB_H, ra_ref[...], rb_ref[...])
    q = z_ref[...] - r
    h = jnp.dot(q, w0_ref[...], preferred_element_type=jnp.float32)
    h = jnp.maximum(h + b0_ref[...], 0.0)
    out_ref[...] = jnp.dot(h, w1_ref[...], preferred_element_type=jnp.float32) + b1_ref[...]


def _decoder(z, ra, rb, w0, b0, w1, b1):
    nb = T // BT_MLP
    return pl.pallas_call(
        _dec_body,
        grid=(nb,),
        in_specs=[
            pl.BlockSpec((BT_MLP, 256), lambda i: (i, 0)),
            pl.BlockSpec((BT_MLP, 256), lambda i: (jnp.minimum(i, NB_H - 1), 0)),
            pl.BlockSpec((BT_MLP, 256), lambda i: (jnp.maximum(i - NB_H, 0), 0)),
            pl.BlockSpec((256, 512), lambda i: (0, 0)),
            pl.BlockSpec((1, 512), lambda i: (0, 0)),
            pl.BlockSpec((512, 768), lambda i: (0, 0)),
            pl.BlockSpec((1, 768), lambda i: (0, 0)),
        ],
        out_specs=pl.BlockSpec((BT_MLP, 768), lambda i: (i, 0)),
        out_shape=jax.ShapeDtypeStruct((T, 768), jnp.float32),
    )(z, ra, rb, w0, b0, w1, b1)


def kernel(x, enc_W0, enc_b0, enc_W1, enc_b1, dec_W0, dec_b0, dec_W1, dec_b1, codebooks):
    B, N, F = x.shape
    xf = x.reshape(T, F)
    z = _encoder(xf, enc_W0, enc_b0.reshape(1, -1), enc_W1, enc_b1.reshape(1, -1))

    cc_all = jnp.sum(codebooks ** 2, axis=-1).reshape(NUM_Q, 1, K)
    cb_flat = codebooks.reshape(NUM_Q * K, D)

    rq = []
    for qt in range(4):
        idx = _vq_argmin_call(0, 0, qt * NB_Q, NB_Q)(z, codebooks, cc_all)
        rq.append(_sc_update(cb_flat, idx.reshape(NW, GCQ), z,
                             nt=TQ, roff=qt * TQ))

    r = [None, None]
    for h in range(2):
        idx, rh = _vq_trans_call()(rq[2 * h], rq[2 * h + 1], codebooks, cc_all)
        r[h] = _sc_update(cb_flat, idx.reshape(NW, GCH), rh)

    for i in range(2, NUM_Q):
        for h in range(2):
            idx = _vq_argmin_call(i, i, 0, NB_H)(r[h], codebooks, cc_all)
            r[h] = _sc_update(cb_flat, idx.reshape(NW, GCH), r[h])

    out = _decoder(z, r[0], r[1], dec_W0, dec_b0.reshape(1, -1),
                   dec_W1, dec_b1.reshape(1, -1))
    return out.reshape(B, N, 768)


# speedup vs baseline: 1.2666x; 1.0093x over previous
import functools

import jax
import jax.numpy as jnp
from jax import lax
from jax.experimental import pallas as pl
from jax.experimental.pallas import tpu as pltpu
from jax.experimental.pallas import tpu_sc as plsc

NUM_Q = 8
K = 8192
D = 256
T = 8192
TH = T // 2
TQ = T // 4
BT_VQ = 512
NB_H = TH // BT_VQ
NB_Q = TQ // BT_VQ
BT_MLP = 1024

SC_CORES = 2
SC_SUBCORES = 16
NW = SC_CORES * SC_SUBCORES
GCH = TH // NW
GCQ = TQ // NW


def _enc_body(x_ref, w0_ref, b0_ref, w1_ref, b1_ref, z_ref):
    h = jnp.dot(x_ref[...], w0_ref[...], preferred_element_type=jnp.float32)
    h = jnp.maximum(h + b0_ref[...], 0.0)
    z_ref[...] = jnp.dot(h, w1_ref[...], preferred_element_type=jnp.float32) + b1_ref[...]


def _encoder(xf, w0, b0, w1, b1):
    nb = T // BT_MLP
    return pl.pallas_call(
        _enc_body,
        grid=(nb,),
        in_specs=[
            pl.BlockSpec((BT_MLP, 768), lambda i: (i, 0)),
            pl.BlockSpec((768, 512), lambda i: (0, 0)),
            pl.BlockSpec((1, 512), lambda i: (0, 0)),
            pl.BlockSpec((512, 256), lambda i: (0, 0)),
            pl.BlockSpec((1, 256), lambda i: (0, 0)),
        ],
        out_specs=pl.BlockSpec((BT_MLP, 256), lambda i: (i, 0)),
        out_shape=jax.ShapeDtypeStruct((T, 256), jnp.float32),
    )(xf, w0, b0, w1, b1)


def _argmin_of(r, cb, cc):
    rc2 = lax.dot_general(r * -2.0, cb, (((1,), (1,)), ((), ())),
                          preferred_element_type=jnp.float32)
    rr = jnp.sum(r * r, axis=1, keepdims=True)
    d = (rr + rc2) + cc
    return jnp.argmin(d, axis=1).astype(jnp.int32)


def _make_vq_body(idx_off):
    def body(r_ref, cb_ref, cc_ref, idx_ref):
        idx_ref[0, 0, :] = _argmin_of(r_ref[...], cb_ref[0], cc_ref[0]) + idx_off
    return body


@functools.lru_cache(maxsize=None)
def _vq_argmin_call(layer, cc_idx, roff, nb):
    return pl.pallas_call(
        _make_vq_body(layer * K),
        grid=(nb,),
        in_specs=[
            pl.BlockSpec((BT_VQ, D), lambda i: (i + roff, 0)),
            pl.BlockSpec((1, K, D), lambda i: (layer, 0, 0)),
            pl.BlockSpec((1, 1, K), lambda i: (cc_idx, 0, 0)),
        ],
        out_specs=pl.BlockSpec((1, 1, BT_VQ), lambda i: (i, 0, 0)),
        out_shape=jax.ShapeDtypeStruct((nb, 1, BT_VQ), jnp.int32),
    )


def _vq_trans_body(rqa_ref, rqb_ref, cb_ref, cc_ref, idx_ref, r_ref):
    r = jnp.where(pl.program_id(0) < NB_Q, rqa_ref[...], rqb_ref[...])
    r_ref[...] = r
    idx_ref[0, 0, :] = _argmin_of(r, cb_ref[0], cc_ref[0]) + K


@functools.lru_cache(maxsize=1)
def _vq_trans_call():
    return pl.pallas_call(
        _vq_trans_body,
        grid=(NB_H,),
        in_specs=[
            pl.BlockSpec((BT_VQ, D), lambda i: (jnp.minimum(i, NB_Q - 1), 0)),
            pl.BlockSpec((BT_VQ, D), lambda i: (jnp.maximum(i - NB_Q, 0), 0)),
            pl.BlockSpec((1, K, D), lambda i: (1, 0, 0)),
            pl.BlockSpec((1, 1, K), lambda i: (1, 0, 0)),
        ],
        out_specs=[
            pl.BlockSpec((1, 1, BT_VQ), lambda i: (i, 0, 0)),
            pl.BlockSpec((BT_VQ, D), lambda i: (i, 0)),
        ],
        out_shape=[
            jax.ShapeDtypeStruct((NB_H, 1, BT_VQ), jnp.int32),
            jax.ShapeDtypeStruct((TH, D), jnp.float32),
        ],
    )


def _make_sc_body(gch, roff):
    def body(cb_hbm, idx_hbm, rp_hbm, out_hbm, idx_v, rows_v, rp_v, sem):
        wid = lax.axis_index("c") * SC_SUBCORES + lax.axis_index("s")
        pltpu.sync_copy(idx_hbm.at[pl.ds(wid, 1)], idx_v)
        gather = pltpu.async_copy(cb_hbm.at[idx_v.at[0]], rows_v, sem)
        pltpu.sync_copy(rp_hbm.at[pl.ds(roff + wid * gch, gch)], rp_v)
        gather.wait()

        def row_fn(i, carry):
            for c in range(D // 16):
                sl = pl.ds(c * 16, 16)
                rp_v[i, sl] = rp_v[i, sl] - rows_v[i, sl]
            return carry

        lax.fori_loop(0, gch, row_fn, 0)
        pltpu.sync_copy(rp_v, out_hbm.at[pl.ds(wid * gch, gch)])
    return body


@functools.lru_cache(maxsize=None)
def _sc_update_call(nt, gch, roff):
    return functools.partial(
        pl.kernel,
        mesh=plsc.VectorSubcoreMesh(core_axis_name="c", subcore_axis_name="s",
                                    num_cores=SC_CORES),
        out_type=jax.ShapeDtypeStruct((nt, D), jnp.float32),
        scratch_types=[
            pltpu.VMEM((1, gch), jnp.int32),
            pltpu.VMEM((gch, D), jnp.float32),
            pltpu.VMEM((gch, D), jnp.float32),
            pltpu.SemaphoreType.DMA,
        ],
    )(_make_sc_body(gch, roff))


def _sc_update(cb_flat, idx2, r_prev, nt=TH, roff=0):
    return _sc_update_call(nt, idx2.shape[1], roff)(cb_flat, idx2, r_prev)


def _dec_body(z_ref, ra_ref, rb_ref, w0_ref, b0_ref, w1_ref, b1_ref, out_ref):
    r = jnp.where(pl.program_id(0) < NB_H, ra_ref[...], rb_ref[...])
    q = z_ref[...] - r
    h = jnp.dot(q, w0_ref[...], preferred_element_type=jnp.float32)
    h = jnp.maximum(h + b0_ref[...], 0.0)
    out_ref[...] = jnp.dot(h, w1_ref[...], preferred_element_type=jnp.float32) + b1_ref[...]


def _decoder(z, ra, rb, w0, b0, w1, b1):
    nb = T // BT_MLP
    return pl.pallas_call(
        _dec_body,
        grid=(nb,),
        in_specs=[
            pl.BlockSpec((BT_MLP, 256), lambda i: (i, 0)),
            pl.BlockSpec((BT_MLP, 256), lambda i: (jnp.minimum(i, NB_H - 1), 0)),
            pl.BlockSpec((BT_MLP, 256), lambda i: (jnp.maximum(i - NB_H, 0), 0)),
            pl.BlockSpec((256, 512), lambda i: (0, 0)),
            pl.BlockSpec((1, 512), lambda i: (0, 0)),
            pl.BlockSpec((512, 768), lambda i: (0, 0)),
            pl.BlockSpec((1, 768), lambda i: (0, 0)),
        ],
        out_specs=pl.BlockSpec((BT_MLP, 768), lambda i: (i, 0)),
        out_shape=jax.ShapeDtypeStruct((T, 768), jnp.float32),
    )(z, ra, rb, w0, b0, w1, b1)


def kernel(x, enc_W0, enc_b0, enc_W1, enc_b1, dec_W0, dec_b0, dec_W1, dec_b1, codebooks):
    B, N, F = x.shape
    xf = x.reshape(T, F)
    z = _encoder(xf, enc_W0, enc_b0.reshape(1, -1), enc_W1, enc_b1.reshape(1, -1))

    cc_all = jnp.sum(codebooks ** 2, axis=-1).reshape(NUM_Q, 1, K)
    cb_flat = codebooks.reshape(NUM_Q * K, D)

    rq = []
    for qt in range(4):
        idx = _vq_argmin_call(0, 0, qt * NB_Q, NB_Q)(z, codebooks, cc_all)
        rq.append(_sc_update(cb_flat, idx.reshape(NW, GCQ), z,
                             nt=TQ, roff=qt * TQ))

    r = [None, None]
    for h in range(2):
        idx, rh = _vq_trans_call()(rq[2 * h], rq[2 * h + 1], codebooks, cc_all)
        r[h] = _sc_update(cb_flat, idx.reshape(NW, GCH), rh)

    for i in range(2, NUM_Q):
        for h in range(2):
            idx = _vq_argmin_call(i, i, 0, NB_H)(r[h], codebooks, cc_all)
            r[h] = _sc_update(cb_flat, idx.reshape(NW, GCH), r[h])

    out = _decoder(z, r[0], r[1], dec_W0, dec_b0.reshape(1, -1),
                   dec_W1, dec_b1.reshape(1, -1))
    return out.reshape(B, N, 768)
